# baseline (speedup 1.0000x reference)
import jax
import jax.numpy as jnp
from jax.experimental import pallas as pl
import jax.experimental.pallas.tpu as pltpu

_CHUNKS = 8


def _gcn2_kernel(x_ref, a_hbm, w1_ref, b1_ref, w2_ref, b2_ref, out_ref,
                 a_vmem, sems):
    n = a_vmem.shape[0]
    bk = n // _CHUNKS
    for i in range(_CHUNKS):
        pltpu.make_async_copy(
            a_hbm.at[pl.ds(i * bk, bk), :],
            a_vmem.at[pl.ds(i * bk, bk), :],
            sems.at[i],
        ).start()

    h = jnp.dot(x_ref[...], w1_ref[...], preferred_element_type=jnp.float32)

    deg_row = jnp.full((1, n), 1.0, dtype=jnp.float32)
    for i in range(_CHUNKS):
        pltpu.make_async_copy(
            a_hbm.at[pl.ds(i * bk, bk), :],
            a_vmem.at[pl.ds(i * bk, bk), :],
            sems.at[i],
        ).wait()
        deg_row += jnp.sum(a_vmem[pl.ds(i * bk, bk), :], axis=0,
                           keepdims=True)
    s_row = jnp.where(deg_row > 0, jax.lax.rsqrt(deg_row), 0.0)
    s = jnp.transpose(s_row)
    a = a_vmem[...]

    def layer(h_lin, b_ref):
        hs = s * h_lin
        m = jax.lax.dot_general(
            a, hs, (((0,), (0,)), ((), ())),
            preferred_element_type=jnp.float32,
        ) + hs
        return jax.nn.relu(s * m + b_ref[...])

    h1 = layer(h, b1_ref)
    h2 = jnp.dot(h1, w2_ref[...], preferred_element_type=jnp.float32)
    out_ref[...] = layer(h2, b2_ref)


def kernel(x, adjacency_matrix, W1, b1, W2, b2):
    n, d_out = x.shape[0], W2.shape[1]
    return pl.pallas_call(
        _gcn2_kernel,
        in_specs=[
            pl.BlockSpec(memory_space=pltpu.MemorySpace.VMEM),
            pl.BlockSpec(memory_space=pltpu.MemorySpace.HBM),
            pl.BlockSpec(memory_space=pltpu.MemorySpace.VMEM),
            pl.BlockSpec(memory_space=pltpu.MemorySpace.VMEM),
            pl.BlockSpec(memory_space=pltpu.MemorySpace.VMEM),
            pl.BlockSpec(memory_space=pltpu.MemorySpace.VMEM),
        ],
        out_specs=pl.BlockSpec(memory_space=pltpu.MemorySpace.VMEM),
        scratch_shapes=[
            pltpu.VMEM((n, n), jnp.float32),
            pltpu.SemaphoreType.DMA((_CHUNKS,)),
        ],
        out_shape=jax.ShapeDtypeStruct((n, d_out), x.dtype),
    )(
        x,
        adjacency_matrix,
        W1,
        b1.reshape(1, -1),
        W2,
        b2.reshape(1, -1),
    )


# speedup vs baseline: 1.2460x; 1.2460x over previous
import jax
import jax.numpy as jnp
from jax.experimental import pallas as pl

_H = 2


def _gcn2_kernel(x_ref, a_ref, w1_ref, b1_ref, w2_ref, b2_ref, out_ref):
    a = a_ref[...]
    n = a.shape[0]
    bk = n // _H
    deg_row = jnp.sum(a, axis=0, keepdims=True) + 1.0
    s_row = jnp.where(deg_row > 0, jax.lax.rsqrt(deg_row), 0.0)
    s = jnp.transpose(s_row)

    def layer(h_lin, b_ref):
        hs = s * h_lin
        out_blocks = []
        for j in range(_H):
            lo, hi = j * bk, (j + 1) * bk
            m = jax.lax.dot_general(
                a[:, lo:hi], hs, (((0,), (0,)), ((), ())),
                preferred_element_type=jnp.float32,
            ) + hs[lo:hi, :]
            out_blocks.append(jax.nn.relu(s[lo:hi, :] * m + b_ref[...]))
        return out_blocks

    h = jnp.dot(x_ref[...], w1_ref[...], preferred_element_type=jnp.float32)
    h1 = layer(h, b1_ref)
    h2 = jnp.concatenate(
        [jnp.dot(blk, w2_ref[...], preferred_element_type=jnp.float32)
         for blk in h1],
        axis=0,
    )
    o = layer(h2, b2_ref)
    for j in range(_H):
        out_ref[pl.ds(j * bk, bk), :] = o[j]


def kernel(x, adjacency_matrix, W1, b1, W2, b2):
    n, d_out = x.shape[0], W2.shape[1]
    return pl.pallas_call(
        _gcn2_kernel,
        out_shape=jax.ShapeDtypeStruct((n, d_out), x.dtype),
    )(
        x,
        adjacency_matrix,
        W1,
        b1.reshape(1, -1),
        W2,
        b2.reshape(1, -1),
    )


# speedup vs baseline: 1.3179x; 1.0577x over previous
import jax
import jax.numpy as jnp
from jax.experimental import pallas as pl

_H = 4


def _gcn2_kernel(x_ref, a_ref, w1_ref, b1_ref, w2_ref, b2_ref, out_ref):
    a = a_ref[...]
    n = a.shape[0]
    bk = n // _H
    deg_row = jnp.sum(a, axis=0, keepdims=True) + 1.0
    s_row = jnp.where(deg_row > 0, jax.lax.rsqrt(deg_row), 0.0)
    s = jnp.transpose(s_row)

    def layer(h_lin, b_ref):
        hs = s * h_lin
        out_blocks = []
        for j in range(_H):
            lo, hi = j * bk, (j + 1) * bk
            m = jax.lax.dot_general(
                a[:, lo:hi], hs, (((0,), (0,)), ((), ())),
                preferred_element_type=jnp.float32,
            ) + hs[lo:hi, :]
            out_blocks.append(jax.nn.relu(s[lo:hi, :] * m + b_ref[...]))
        return out_blocks

    h = jnp.dot(x_ref[...], w1_ref[...], preferred_element_type=jnp.float32)
    h1 = layer(h, b1_ref)
    h2 = jnp.concatenate(
        [jnp.dot(blk, w2_ref[...], preferred_element_type=jnp.float32)
         for blk in h1],
        axis=0,
    )
    o = layer(h2, b2_ref)
    for j in range(_H):
        out_ref[pl.ds(j * bk, bk), :] = o[j]


def kernel(x, adjacency_matrix, W1, b1, W2, b2):
    n, d_out = x.shape[0], W2.shape[1]
    return pl.pallas_call(
        _gcn2_kernel,
        out_shape=jax.ShapeDtypeStruct((n, d_out), x.dtype),
    )(
        x,
        adjacency_matrix,
        W1,
        b1.reshape(1, -1),
        W2,
        b2.reshape(1, -1),
    )
